# baseline (speedup 1.0000x reference)
import jax
import jax.numpy as jnp
from jax import lax
from jax.experimental import pallas as pl
from jax.experimental.pallas import tpu as pltpu
from jax.experimental.pallas import tpu_sc as plsc

BATCH = 4096
N_FIELDS = 26
N_DENSE = 13
VOCAB = 100000
EMBED = 64
LANES = 16

NC = 2
NS = 16

W = 256
NCHUNK = VOCAB // W + 1
NFULL = NCHUNK - 1
TAIL0 = (VOCAB // 128) * 128
TAIL_MAIN0 = NFULL * W
NROWS = N_FIELDS * BATCH
FLUSH = 128
DUMP0 = NROWS
OUT3_ROWS = NROWS + 32 * FLUSH
HBUF = 400


def _body(sidx_hbm, tablesT_hbm, tail_hbm, out_hbm,
          vidx_v, sorted_v, hist_v, cur_v, off_s,
          buf0, buf1, buf2, stag_v, oidx_v, sem0, sem1, sem2, wsem):
    wid = lax.axis_index("s") * NC + lax.axis_index("c")

    @pl.when(wid < N_FIELDS)
    def _active():
        f = wid
        lane = lax.iota(jnp.int32, LANES)
        zeros16 = jnp.zeros((LANES,), jnp.int32)
        ones16 = jnp.ones((LANES,), jnp.int32)

        pltpu.sync_copy(
            sidx_hbm.at[pl.ds(pl.multiple_of(f * BATCH, BATCH), BATCH)],
            vidx_v)

        def _issue(c, buf, sem):
            col0 = pl.multiple_of(c * W, W)
            pltpu.async_copy(tablesT_hbm.at[f, :, pl.ds(col0, W)], buf, sem)

        def _wait(buf, sem):
            pltpu.make_async_copy(tablesT_hbm.at[f, :, pl.ds(0, W)],
                                  buf, sem).wait()

        _issue(0, buf0, sem0)
        _issue(1, buf1, sem1)
        _issue(2, buf2, sem2)

        for t in range(HBUF // LANES):
            hist_v[pl.ds(t * LANES, LANES)] = zeros16

        def _hist(t, carry):
            v16 = plsc.load_gather(vidx_v, [t * LANES + lane])
            plsc.addupdate_scatter(hist_v, [v16 >> 8], ones16)
            return carry
        lax.fori_loop(0, BATCH // LANES, _hist, 0)

        carry = jnp.int32(0)
        for t in range(HBUF // LANES):
            h16 = hist_v[pl.ds(t * LANES, LANES)]
            incl = plsc.cumsum(h16)
            excl = incl - h16 + carry
            cur_v[pl.ds(t * LANES, LANES)] = excl
            for i in range(LANES):
                off_s[t * LANES + i] = excl[i]
            carry = carry + incl[LANES - 1]

        def _place(t, carry):
            kv = t * LANES + lane
            v16 = plsc.load_gather(vidx_v, [kv])
            c16 = v16 >> 8
            rec16 = (kv << 8) | (v16 & (W - 1))
            dup16, _last = plsc.scan_count(c16)
            base16 = plsc.load_gather(cur_v, [c16])
            plsc.store_scatter(sorted_v, [base16 + dup16 - 1], rec16)
            plsc.addupdate_scatter(cur_v, [c16], ones16)
            return carry
        lax.fori_loop(0, BATCH // LANES, _place, 0)

        dump = DUMP0 + wid * FLUSH
        dump_lane = dump + lane

        def _reset_oidx():
            for t in range(FLUSH // LANES):
                oidx_v[pl.ds(t * LANES, LANES)] = dump + t * LANES + lane
        _reset_oidx()

        def _flush():
            pltpu.async_copy(stag_v, out_hbm.at[oidx_v], wsem).wait()
            _reset_oidx()

        def _process(c, buf, j):
            n0 = off_s[c]
            n1 = off_s[c + 1]
            ngroups = (n1 - n0 + LANES - 1) >> 4

            def _group(g, j):
                @pl.when(j == FLUSH)
                def _():
                    _flush()
                j = lax.select(j == FLUSH, 0, j)
                kv = n0 + g * LANES + lane
                kidx = jnp.minimum(kv, n1 - 1)
                rec16 = plsc.load_gather(sorted_v, [kidx])
                rowid16 = jnp.where(kv < n1,
                                    (rec16 >> 8) * N_FIELDS + f, dump_lane)
                plsc.store_scatter(oidx_v, [j + lane], rowid16)
                for i in range(LANES):
                    col = rec16[i] & (W - 1)
                    cols = jnp.full((LANES,), col, jnp.int32)
                    j16 = jnp.full((LANES,), j + i, jnp.int32)
                    for t in range(EMBED // LANES):
                        vals = plsc.load_gather(buf, [lane + t * LANES, cols])
                        plsc.store_scatter(stag_v, [j16, lane + t * LANES],
                                           vals)
                return j + LANES
            return lax.fori_loop(0, ngroups, _group, j)

        def _trip(i, j):
            c0 = 3 * i
            _wait(buf0, sem0)
            j = _process(c0, buf0, j)

            @pl.when(c0 + 3 < NFULL)
            def _():
                _issue(c0 + 3, buf0, sem0)
            _wait(buf1, sem1)
            j = _process(c0 + 1, buf1, j)

            @pl.when(c0 + 4 < NFULL)
            def _():
                _issue(c0 + 4, buf1, sem1)
            _wait(buf2, sem2)
            j = _process(c0 + 2, buf2, j)

            @pl.when(c0 + 5 < NFULL)
            def _():
                _issue(c0 + 5, buf2, sem2)
            return j

        j = lax.fori_loop(0, NFULL // 3, _trip, 0)

        pltpu.sync_copy(tablesT_hbm.at[f, :, pl.ds(TAIL_MAIN0, 128)],
                        buf0.at[:, pl.ds(0, 128)])
        pltpu.sync_copy(tail_hbm.at[f], buf0.at[:, pl.ds(128, 128)])
        j = _process(NFULL, buf0, j)
        _flush()


@jax.jit
def _sc_embed(sidxT, tablesT, tailT):
    mesh = plsc.VectorSubcoreMesh(core_axis_name="c", subcore_axis_name="s")
    fn = pl.kernel(
        _body,
        mesh=mesh,
        compiler_params=pltpu.CompilerParams(needs_layout_passes=False),
        out_type=jax.ShapeDtypeStruct((OUT3_ROWS, 2 * EMBED), jnp.float32),
        scratch_types=[
            pltpu.VMEM((BATCH,), jnp.int32),
            pltpu.VMEM((BATCH,), jnp.int32),
            pltpu.VMEM((HBUF,), jnp.int32),
            pltpu.VMEM((HBUF,), jnp.int32),
            pltpu.SMEM((HBUF,), jnp.int32),
            pltpu.VMEM((EMBED, W), jnp.float32),
            pltpu.VMEM((EMBED, W), jnp.float32),
            pltpu.VMEM((EMBED, W), jnp.float32),
            pltpu.VMEM((FLUSH, 2 * EMBED), jnp.float32),
            pltpu.VMEM((FLUSH,), jnp.int32),
            pltpu.SemaphoreType.DMA,
            pltpu.SemaphoreType.DMA,
            pltpu.SemaphoreType.DMA,
            pltpu.SemaphoreType.DMA,
        ],
    )
    return fn(sidxT, tablesT, tailT)


def kernel(sparse_indices, dense_values, tables):
    tablesT = tables.transpose(0, 2, 1)
    tailT = tables[:, TAIL0:, :].transpose(0, 2, 1)
    tailT = jnp.pad(tailT, ((0, 0), (0, 0), (0, 128 - (VOCAB - TAIL0))))
    sidxT = sparse_indices.T.reshape(-1)
    out3 = _sc_embed(sidxT, tablesT, tailT)
    emb = out3[:NROWS, :EMBED].reshape(BATCH, N_FIELDS * EMBED)
    return jnp.concatenate([emb, dense_values], axis=1)


# speedup vs baseline: 1.1524x; 1.1524x over previous
import jax
import jax.numpy as jnp
from jax import lax
from jax.experimental import pallas as pl
from jax.experimental.pallas import tpu as pltpu
from jax.experimental.pallas import tpu_sc as plsc

BATCH = 4096
N_FIELDS = 26
N_DENSE = 13
VOCAB = 100000
EMBED = 64
LANES = 16

NC = 2
NS = 16

W = 256
NCHUNK = VOCAB // W + 1
NFULL = NCHUNK - 1
TAIL0 = (VOCAB // 128) * 128
TAIL_MAIN0 = NFULL * W
NROWS = N_FIELDS * BATCH
FLUSH = 128
DUMP0 = NROWS
OUT3_ROWS = NROWS + 32 * FLUSH
HBUF = 400


def _body(sidx_hbm, tablesT_hbm, tail_hbm, out_hbm,
          vidx_v, sorted_v, hist_v, cur_v, off_s,
          buf0, buf1, buf2, stag_v, oidx_v, sem0, sem1, sem2, wsem):
    wid = lax.axis_index("s") * NC + lax.axis_index("c")

    @pl.when(wid < N_FIELDS)
    def _active():
        f = wid
        lane = lax.iota(jnp.int32, LANES)
        zeros16 = jnp.zeros((LANES,), jnp.int32)
        ones16 = jnp.ones((LANES,), jnp.int32)

        pltpu.sync_copy(
            sidx_hbm.at[pl.ds(pl.multiple_of(f * BATCH, BATCH), BATCH)],
            vidx_v)

        def _issue(c, buf, sem):
            col0 = pl.multiple_of(c * W, W)
            pltpu.async_copy(tablesT_hbm.at[f, :, pl.ds(col0, W)], buf, sem)

        def _wait(buf, sem):
            pltpu.make_async_copy(tablesT_hbm.at[f, :, pl.ds(0, W)],
                                  buf, sem).wait()

        _issue(0, buf0, sem0)
        _issue(1, buf1, sem1)
        _issue(2, buf2, sem2)

        for t in range(HBUF // LANES):
            hist_v[pl.ds(t * LANES, LANES)] = zeros16

        def _hist(t, carry):
            v16 = plsc.load_gather(vidx_v, [t * LANES + lane])
            plsc.addupdate_scatter(hist_v, [v16 >> 8], ones16)
            return carry
        lax.fori_loop(0, BATCH // LANES, _hist, 0)

        carry = jnp.int32(0)
        for t in range(HBUF // LANES):
            h16 = hist_v[pl.ds(t * LANES, LANES)]
            incl = plsc.cumsum(h16)
            excl = incl - h16 + carry
            cur_v[pl.ds(t * LANES, LANES)] = excl
            for i in range(LANES):
                off_s[t * LANES + i] = excl[i]
            carry = carry + incl[LANES - 1]

        def _place(t, carry):
            kv = t * LANES + lane
            v16 = plsc.load_gather(vidx_v, [kv])
            c16 = v16 >> 8
            rec16 = (kv << 8) | (v16 & (W - 1))
            dup16, _last = plsc.scan_count(c16)
            base16 = plsc.load_gather(cur_v, [c16])
            plsc.store_scatter(sorted_v, [base16 + dup16 - 1], rec16)
            plsc.addupdate_scatter(cur_v, [c16], ones16)
            return carry
        lax.fori_loop(0, BATCH // LANES, _place, 0)

        dump = DUMP0 + wid * FLUSH
        dump_lane = dump + lane

        def _reset_oidx(q16):
            for t in range(FLUSH // LANES):
                plsc.store_scatter(oidx_v, [q16, t * LANES + lane],
                                   dump + t * LANES + lane)
        _reset_oidx(zeros16)
        _reset_oidx(ones16)

        def _drain():
            pltpu.make_async_copy(stag_v.at[0], out_hbm.at[oidx_v.at[0]],
                                  wsem).wait()

        pltpu.async_copy(stag_v.at[1], out_hbm.at[oidx_v.at[1]], wsem)

        def _flush(p):
            _drain()
            _reset_oidx(jnp.full((LANES,), 1 - p, jnp.int32))
            pltpu.async_copy(stag_v.at[p], out_hbm.at[oidx_v.at[p]], wsem)

        def _process(c, buf, jp):
            n0 = off_s[c]
            n1 = off_s[c + 1]
            ngroups = (n1 - n0 + LANES - 1) >> 4

            def _group(g, jp):
                j, p = jp
                full = j == FLUSH

                @pl.when(full)
                def _():
                    _flush(p)
                j = lax.select(full, 0, j)
                p = lax.select(full, 1 - p, p)
                p16 = jnp.full((LANES,), p, jnp.int32)
                kv = n0 + g * LANES + lane
                kidx = jnp.minimum(kv, n1 - 1)
                rec16 = plsc.load_gather(sorted_v, [kidx])
                rowid16 = jnp.where(kv < n1, f * BATCH + (rec16 >> 8),
                                    dump_lane)
                plsc.store_scatter(oidx_v, [p16, j + lane], rowid16)
                for i in range(LANES):
                    col = rec16[i] & (W - 1)
                    cols = jnp.full((LANES,), col, jnp.int32)
                    j16 = jnp.full((LANES,), j + i, jnp.int32)
                    for t in range(EMBED // LANES):
                        vals = plsc.load_gather(buf, [lane + t * LANES, cols])
                        plsc.store_scatter(stag_v,
                                           [p16, j16, lane + t * LANES], vals)
                return j + LANES, p
            return lax.fori_loop(0, ngroups, _group, jp)

        def _trip(i, jp):
            c0 = 3 * i
            _wait(buf0, sem0)
            jp = _process(c0, buf0, jp)

            @pl.when(c0 + 3 < NFULL)
            def _():
                _issue(c0 + 3, buf0, sem0)
            _wait(buf1, sem1)
            jp = _process(c0 + 1, buf1, jp)

            @pl.when(c0 + 4 < NFULL)
            def _():
                _issue(c0 + 4, buf1, sem1)
            _wait(buf2, sem2)
            jp = _process(c0 + 2, buf2, jp)

            @pl.when(c0 + 5 < NFULL)
            def _():
                _issue(c0 + 5, buf2, sem2)
            return jp

        jp = lax.fori_loop(0, NFULL // 3, _trip,
                           (jnp.int32(0), jnp.int32(0)))

        pltpu.sync_copy(tablesT_hbm.at[f, :, pl.ds(TAIL_MAIN0, 128)],
                        buf0.at[:, pl.ds(0, 128)])
        pltpu.sync_copy(tail_hbm.at[f], buf0.at[:, pl.ds(128, 128)])
        j, p = _process(NFULL, buf0, jp)
        _flush(p)
        _drain()


@jax.jit
def _sc_embed(sidxT, tablesT, tailT):
    mesh = plsc.VectorSubcoreMesh(core_axis_name="c", subcore_axis_name="s")
    fn = pl.kernel(
        _body,
        mesh=mesh,
        compiler_params=pltpu.CompilerParams(needs_layout_passes=False),
        out_type=jax.ShapeDtypeStruct((OUT3_ROWS, 2 * EMBED), jnp.float32),
        scratch_types=[
            pltpu.VMEM((BATCH,), jnp.int32),
            pltpu.VMEM((BATCH,), jnp.int32),
            pltpu.VMEM((HBUF,), jnp.int32),
            pltpu.VMEM((HBUF,), jnp.int32),
            pltpu.SMEM((HBUF,), jnp.int32),
            pltpu.VMEM((EMBED, W), jnp.float32),
            pltpu.VMEM((EMBED, W), jnp.float32),
            pltpu.VMEM((EMBED, W), jnp.float32),
            pltpu.VMEM((2, FLUSH, 2 * EMBED), jnp.float32),
            pltpu.VMEM((2, FLUSH), jnp.int32),
            pltpu.SemaphoreType.DMA,
            pltpu.SemaphoreType.DMA,
            pltpu.SemaphoreType.DMA,
            pltpu.SemaphoreType.DMA,
        ],
    )
    return fn(sidxT, tablesT, tailT)


def kernel(sparse_indices, dense_values, tables):
    tablesT = tables.transpose(0, 2, 1)
    tailT = tables[:, TAIL0:, :].transpose(0, 2, 1)
    tailT = jnp.pad(tailT, ((0, 0), (0, 0), (0, 128 - (VOCAB - TAIL0))))
    sidxT = sparse_indices.T.reshape(-1)
    out3 = _sc_embed(sidxT, tablesT, tailT)
    emb = out3[:NROWS, :EMBED].reshape(N_FIELDS, BATCH, EMBED)
    emb = emb.transpose(1, 0, 2).reshape(BATCH, N_FIELDS * EMBED)
    return jnp.concatenate([emb, dense_values], axis=1)


# speedup vs baseline: 1.2618x; 1.0950x over previous
import jax
import jax.numpy as jnp
from jax import lax
from jax.experimental import pallas as pl
from jax.experimental.pallas import tpu as pltpu
from jax.experimental.pallas import tpu_sc as plsc

BATCH = 4096
N_FIELDS = 26
N_DENSE = 13
VOCAB = 100000
EMBED = 64
LANES = 16

NC = 2
NS = 16

W = 256
NCHUNK = VOCAB // W + 1
NFULL = NCHUNK - 1
TAIL0 = (VOCAB // 128) * 128
TAIL_MAIN0 = NFULL * W
NROWS = N_FIELDS * BATCH
FLUSH = 128
DUMP0 = NROWS
OUT3_ROWS = NROWS + 32 * FLUSH
HBUF = 400
NW = NC * NS
TOTALC = N_FIELDS * NCHUNK


def _body(sidx_hbm, tablesT_hbm, tail_hbm, out_hbm,
          vidx_v, sorted_v, hist_v, cur_v, off_s,
          buf0, buf1, buf2, stag_v, oidx_v, sem0, sem1, sem2, wsem):
    wid = lax.axis_index("s") * NC + lax.axis_index("c")
    lane = lax.iota(jnp.int32, LANES)
    zeros16 = jnp.zeros((LANES,), jnp.int32)
    ones16 = jnp.ones((LANES,), jnp.int32)
    bufs = ((buf0, sem0), (buf1, sem1), (buf2, sem2))

    dump = DUMP0 + wid * FLUSH
    dump_lane = dump + lane

    def _reset_oidx(q16):
        for t in range(FLUSH // LANES):
            plsc.store_scatter(oidx_v, [q16, t * LANES + lane],
                               dump + t * LANES + lane)
    _reset_oidx(zeros16)
    _reset_oidx(ones16)

    def _drain():
        pltpu.make_async_copy(stag_v.at[0], out_hbm.at[oidx_v.at[0]],
                              wsem).wait()

    pltpu.async_copy(stag_v.at[1], out_hbm.at[oidx_v.at[1]], wsem)

    def _flush(p):
        _drain()
        _reset_oidx(jnp.full((LANES,), 1 - p, jnp.int32))
        pltpu.async_copy(stag_v.at[p], out_hbm.at[oidx_v.at[p]], wsem)

    def _segment(f, clo, chi, jp):
        pltpu.sync_copy(
            sidx_hbm.at[pl.ds(pl.multiple_of(f * BATCH, BATCH), BATCH)],
            vidx_v)

        def _issue(c, buf, sem):
            @pl.when(c < NFULL)
            def _():
                col0 = pl.multiple_of(c * W, W)
                pltpu.async_copy(tablesT_hbm.at[f, :, pl.ds(col0, W)],
                                 buf, sem)

            @pl.when(c == NFULL)
            def _():
                pltpu.async_copy(tablesT_hbm.at[f, :, pl.ds(TAIL_MAIN0, 128)],
                                 buf.at[:, pl.ds(0, 128)], sem)
                pltpu.async_copy(tail_hbm.at[f],
                                 buf.at[:, pl.ds(128, 128)], sem)

        def _wait(buf, sem):
            pltpu.make_async_copy(tablesT_hbm.at[f, :, pl.ds(0, W)],
                                  buf, sem).wait()

        for k, (buf, sem) in enumerate(bufs):
            @pl.when(clo + k < chi)
            def _(k=k, buf=buf, sem=sem):
                _issue(clo + k, buf, sem)

        for t in range(HBUF // LANES):
            hist_v[pl.ds(t * LANES, LANES)] = zeros16

        def _hist(t, carry):
            v16 = plsc.load_gather(vidx_v, [t * LANES + lane])
            plsc.addupdate_scatter(hist_v, [v16 >> 8], ones16)
            return carry
        lax.fori_loop(0, BATCH // LANES, _hist, 0)

        carry = jnp.int32(0)
        for t in range(HBUF // LANES):
            h16 = hist_v[pl.ds(t * LANES, LANES)]
            incl = plsc.cumsum(h16)
            excl = incl - h16 + carry
            cur_v[pl.ds(t * LANES, LANES)] = excl
            for i in range(LANES):
                off_s[t * LANES + i] = excl[i]
            carry = carry + incl[LANES - 1]

        def _place(t, carry):
            kv = t * LANES + lane
            v16 = plsc.load_gather(vidx_v, [kv])
            c16 = v16 >> 8
            rec16 = (kv << 8) | (v16 & (W - 1))
            dup16, _last = plsc.scan_count(c16)
            base16 = plsc.load_gather(cur_v, [c16])
            plsc.store_scatter(sorted_v, [base16 + dup16 - 1], rec16)
            plsc.addupdate_scatter(cur_v, [c16], ones16)
            return carry
        lax.fori_loop(0, BATCH // LANES, _place, 0)

        def _process(n0, n1, buf, jp):
            ngroups = (n1 - n0 + LANES - 1) >> 4

            def _group(g, jp):
                j, p = jp
                full = j == FLUSH

                @pl.when(full)
                def _():
                    _flush(p)
                j = lax.select(full, 0, j)
                p = lax.select(full, 1 - p, p)
                p16 = jnp.full((LANES,), p, jnp.int32)
                kv = n0 + g * LANES + lane
                kidx = jnp.minimum(kv, n1 - 1)
                rec16 = plsc.load_gather(sorted_v, [kidx])
                rowid16 = jnp.where(kv < n1, f * BATCH + (rec16 >> 8),
                                    dump_lane)
                plsc.store_scatter(oidx_v, [p16, j + lane], rowid16)
                for i in range(LANES):
                    col = rec16[i] & (W - 1)
                    cols = jnp.full((LANES,), col, jnp.int32)
                    j16 = jnp.full((LANES,), j + i, jnp.int32)
                    for t in range(EMBED // LANES):
                        vals = plsc.load_gather(buf, [lane + t * LANES, cols])
                        plsc.store_scatter(stag_v,
                                           [p16, j16, lane + t * LANES], vals)
                return j + LANES, p
            return lax.fori_loop(0, ngroups, _group, jp)

        ntrips = (chi - clo + 2) // 3

        def _trip(i, jp):
            c0 = clo + 3 * i
            for k, (buf, sem) in enumerate(bufs):
                c = c0 + k
                live = c < chi

                @pl.when(live)
                def _(buf=buf, sem=sem):
                    _wait(buf, sem)
                cc = jnp.minimum(c, NCHUNK - 1)
                n0 = lax.select(live, off_s[cc], 0)
                n1 = lax.select(live, off_s[cc + 1], 0)
                jp = _process(n0, n1, buf, jp)

                @pl.when(c + 3 < chi)
                def _(c=c, buf=buf, sem=sem):
                    _issue(c + 3, buf, sem)
            return jp

        return lax.fori_loop(0, ntrips, _trip, jp)

    g0 = (wid * TOTALC) // NW
    g1 = ((wid + 1) * TOTALC) // NW
    fA = g0 // NCHUNK
    cA0 = g0 % NCHUNK
    fZ = (g1 - 1) // NCHUNK
    cZ1 = (g1 - 1) % NCHUNK + 1
    two = fZ > fA
    eA = lax.select(two, jnp.int32(NCHUNK), cZ1)
    cB0 = jnp.int32(0)
    eB = lax.select(two, cZ1, jnp.int32(0))

    jp = (jnp.int32(0), jnp.int32(0))
    jp = _segment(fA, cA0, eA, jp)
    jp = _segment(fZ, cB0, eB, jp)
    j, p = jp
    _flush(p)
    _drain()


@jax.jit
def _sc_embed(sidxT, tablesT, tailT):
    mesh = plsc.VectorSubcoreMesh(core_axis_name="c", subcore_axis_name="s")
    fn = pl.kernel(
        _body,
        mesh=mesh,
        compiler_params=pltpu.CompilerParams(needs_layout_passes=False),
        out_type=jax.ShapeDtypeStruct((OUT3_ROWS, 2 * EMBED), jnp.float32),
        scratch_types=[
            pltpu.VMEM((BATCH,), jnp.int32),
            pltpu.VMEM((BATCH,), jnp.int32),
            pltpu.VMEM((HBUF,), jnp.int32),
            pltpu.VMEM((HBUF,), jnp.int32),
            pltpu.SMEM((HBUF,), jnp.int32),
            pltpu.VMEM((EMBED, W), jnp.float32),
            pltpu.VMEM((EMBED, W), jnp.float32),
            pltpu.VMEM((EMBED, W), jnp.float32),
            pltpu.VMEM((2, FLUSH, 2 * EMBED), jnp.float32),
            pltpu.VMEM((2, FLUSH), jnp.int32),
            pltpu.SemaphoreType.DMA,
            pltpu.SemaphoreType.DMA,
            pltpu.SemaphoreType.DMA,
            pltpu.SemaphoreType.DMA,
        ],
    )
    return fn(sidxT, tablesT, tailT)


def kernel(sparse_indices, dense_values, tables):
    tablesT = tables.transpose(0, 2, 1)
    tailT = tables[:, TAIL0:, :].transpose(0, 2, 1)
    tailT = jnp.pad(tailT, ((0, 0), (0, 0), (0, 128 - (VOCAB - TAIL0))))
    sidxT = sparse_indices.T.reshape(-1)
    out3 = _sc_embed(sidxT, tablesT, tailT)
    emb = out3[:NROWS, :EMBED].reshape(N_FIELDS, BATCH, EMBED)
    emb = emb.transpose(1, 0, 2).reshape(BATCH, N_FIELDS * EMBED)
    return jnp.concatenate([emb, dense_values], axis=1)


# speedup vs baseline: 1.2962x; 1.0272x over previous
import jax
import jax.numpy as jnp
from jax import lax
from jax.experimental import pallas as pl
from jax.experimental.pallas import tpu as pltpu
from jax.experimental.pallas import tpu_sc as plsc

BATCH = 4096
N_FIELDS = 26
N_DENSE = 13
VOCAB = 100000
EMBED = 64
LANES = 16

NC = 2
NS = 16

W = 256
NCHUNK = VOCAB // W + 1
NFULL = NCHUNK - 1
TAIL0 = (VOCAB // 128) * 128
TAIL_MAIN0 = NFULL * W
NROWS = N_FIELDS * BATCH
FLUSH = 128
DUMP0 = NROWS
OUT3_ROWS = NROWS + 32 * FLUSH
HBUF = 400
NW = NC * NS
TOTALC = N_FIELDS * NCHUNK


def _body(sidx_hbm, tablesT_hbm, tail_hbm, out_hbm,
          vidx_v, sorted_v, hist_v, cur_v, off_s,
          buf0, buf1, buf2, buf3, stag_v, oidx_v,
          sem0, sem1, sem2, sem3, wsem):
    wid = lax.axis_index("s") * NC + lax.axis_index("c")
    lane = lax.iota(jnp.int32, LANES)
    zeros16 = jnp.zeros((LANES,), jnp.int32)
    ones16 = jnp.ones((LANES,), jnp.int32)
    bufs = ((buf0, sem0), (buf1, sem1), (buf2, sem2), (buf3, sem3))

    dump = DUMP0 + wid * FLUSH
    dump_lane = dump + lane

    def _reset_oidx(q16):
        for t in range(FLUSH // LANES):
            plsc.store_scatter(oidx_v, [q16, t * LANES + lane],
                               dump + t * LANES + lane)
    _reset_oidx(zeros16)
    _reset_oidx(ones16)

    def _drain():
        pltpu.make_async_copy(stag_v.at[0], out_hbm.at[oidx_v.at[0]],
                              wsem).wait()

    pltpu.async_copy(stag_v.at[1], out_hbm.at[oidx_v.at[1]], wsem)

    def _flush(p):
        _drain()
        _reset_oidx(jnp.full((LANES,), 1 - p, jnp.int32))
        pltpu.async_copy(stag_v.at[p], out_hbm.at[oidx_v.at[p]], wsem)

    def _segment(f, clo, chi, jp):
        pltpu.sync_copy(
            sidx_hbm.at[pl.ds(pl.multiple_of(f * BATCH, BATCH), BATCH)],
            vidx_v)

        def _issue(c, buf, sem):
            @pl.when(c < NFULL)
            def _():
                col0 = pl.multiple_of(c * W, W)
                pltpu.async_copy(tablesT_hbm.at[f, :, pl.ds(col0, W)],
                                 buf, sem)

            @pl.when(c == NFULL)
            def _():
                pltpu.async_copy(tablesT_hbm.at[f, :, pl.ds(TAIL_MAIN0, 128)],
                                 buf.at[:, pl.ds(0, 128)], sem)
                pltpu.async_copy(tail_hbm.at[f],
                                 buf.at[:, pl.ds(128, 128)], sem)

        def _wait(buf, sem):
            pltpu.make_async_copy(tablesT_hbm.at[f, :, pl.ds(0, W)],
                                  buf, sem).wait()

        for k, (buf, sem) in enumerate(bufs):
            @pl.when(clo + k < chi)
            def _(k=k, buf=buf, sem=sem):
                _issue(clo + k, buf, sem)

        for t in range(HBUF // LANES):
            hist_v[pl.ds(t * LANES, LANES)] = zeros16

        def _hist(t, carry):
            v16 = plsc.load_gather(vidx_v, [t * LANES + lane])
            plsc.addupdate_scatter(hist_v, [v16 >> 8], ones16)
            return carry
        lax.fori_loop(0, BATCH // LANES, _hist, 0)

        carry = jnp.int32(0)
        for t in range(HBUF // LANES):
            h16 = hist_v[pl.ds(t * LANES, LANES)]
            incl = plsc.cumsum(h16)
            excl = incl - h16 + carry
            cur_v[pl.ds(t * LANES, LANES)] = excl
            for i in range(LANES):
                off_s[t * LANES + i] = excl[i]
            carry = carry + incl[LANES - 1]

        def _place(t, carry):
            kv = t * LANES + lane
            v16 = plsc.load_gather(vidx_v, [kv])
            c16 = v16 >> 8
            rec16 = (kv << 8) | (v16 & (W - 1))
            dup16, _last = plsc.scan_count(c16)
            base16 = plsc.load_gather(cur_v, [c16])
            plsc.store_scatter(sorted_v, [base16 + dup16 - 1], rec16)
            plsc.addupdate_scatter(cur_v, [c16], ones16)
            return carry
        lax.fori_loop(0, BATCH // LANES, _place, 0)

        def _process(n0, n1, buf, jp):
            ngroups = (n1 - n0 + LANES - 1) >> 4

            def _group(g, jp):
                j, p = jp
                full = j == FLUSH

                @pl.when(full)
                def _():
                    _flush(p)
                j = lax.select(full, 0, j)
                p = lax.select(full, 1 - p, p)
                p16 = jnp.full((LANES,), p, jnp.int32)
                kv = n0 + g * LANES + lane
                kidx = jnp.minimum(kv, n1 - 1)
                rec16 = plsc.load_gather(sorted_v, [kidx])
                rowid16 = jnp.where(kv < n1, f * BATCH + (rec16 >> 8),
                                    dump_lane)
                plsc.store_scatter(oidx_v, [p16, j + lane], rowid16)
                for i in range(LANES):
                    col = rec16[i] & (W - 1)
                    cols = jnp.full((LANES,), col, jnp.int32)
                    j16 = jnp.full((LANES,), j + i, jnp.int32)
                    for t in range(EMBED // LANES):
                        vals = plsc.load_gather(buf, [lane + t * LANES, cols])
                        plsc.store_scatter(stag_v,
                                           [p16, j16, lane + t * LANES], vals)
                return j + LANES, p
            return lax.fori_loop(0, ngroups, _group, jp)

        ntrips = (chi - clo + 3) // 4

        def _trip(i, jp):
            c0 = clo + 4 * i
            for k, (buf, sem) in enumerate(bufs):
                c = c0 + k
                live = c < chi

                @pl.when(live)
                def _(buf=buf, sem=sem):
                    _wait(buf, sem)
                cc = jnp.minimum(c, NCHUNK - 1)
                n0 = lax.select(live, off_s[cc], 0)
                n1 = lax.select(live, off_s[cc + 1], 0)
                jp = _process(n0, n1, buf, jp)

                @pl.when(c + 4 < chi)
                def _(c=c, buf=buf, sem=sem):
                    _issue(c + 4, buf, sem)
            return jp

        return lax.fori_loop(0, ntrips, _trip, jp)

    g0 = (wid * TOTALC) // NW
    g1 = ((wid + 1) * TOTALC) // NW
    fA = g0 // NCHUNK
    cA0 = g0 % NCHUNK
    fZ = (g1 - 1) // NCHUNK
    cZ1 = (g1 - 1) % NCHUNK + 1
    two = fZ > fA
    eA = lax.select(two, jnp.int32(NCHUNK), cZ1)
    cB0 = jnp.int32(0)
    eB = lax.select(two, cZ1, jnp.int32(0))

    jp = (jnp.int32(0), jnp.int32(0))
    jp = _segment(fA, cA0, eA, jp)
    jp = _segment(fZ, cB0, eB, jp)
    j, p = jp
    _flush(p)
    _drain()


@jax.jit
def _sc_embed(sidxT, tablesT, tailT):
    mesh = plsc.VectorSubcoreMesh(core_axis_name="c", subcore_axis_name="s")
    fn = pl.kernel(
        _body,
        mesh=mesh,
        compiler_params=pltpu.CompilerParams(needs_layout_passes=False),
        out_type=jax.ShapeDtypeStruct((OUT3_ROWS, 2 * EMBED), jnp.float32),
        scratch_types=[
            pltpu.VMEM((BATCH,), jnp.int32),
            pltpu.VMEM((BATCH,), jnp.int32),
            pltpu.VMEM((HBUF,), jnp.int32),
            pltpu.VMEM((HBUF,), jnp.int32),
            pltpu.SMEM((HBUF,), jnp.int32),
            pltpu.VMEM((EMBED, W), jnp.float32),
            pltpu.VMEM((EMBED, W), jnp.float32),
            pltpu.VMEM((EMBED, W), jnp.float32),
            pltpu.VMEM((EMBED, W), jnp.float32),
            pltpu.VMEM((2, FLUSH, 2 * EMBED), jnp.float32),
            pltpu.VMEM((2, FLUSH), jnp.int32),
            pltpu.SemaphoreType.DMA,
            pltpu.SemaphoreType.DMA,
            pltpu.SemaphoreType.DMA,
            pltpu.SemaphoreType.DMA,
            pltpu.SemaphoreType.DMA,
        ],
    )
    return fn(sidxT, tablesT, tailT)


def kernel(sparse_indices, dense_values, tables):
    tablesT = tables.transpose(0, 2, 1)
    tailT = tables[:, TAIL0:, :].transpose(0, 2, 1)
    tailT = jnp.pad(tailT, ((0, 0), (0, 0), (0, 128 - (VOCAB - TAIL0))))
    sidxT = sparse_indices.T.reshape(-1)
    out3 = _sc_embed(sidxT, tablesT, tailT)
    emb = out3[:NROWS, :EMBED].reshape(N_FIELDS, BATCH, EMBED)
    emb = emb.transpose(1, 0, 2).reshape(BATCH, N_FIELDS * EMBED)
    return jnp.concatenate([emb, dense_values], axis=1)


# speedup vs baseline: 1.3531x; 1.0439x over previous
import jax
import jax.numpy as jnp
from jax import lax
from jax.experimental import pallas as pl
from jax.experimental.pallas import tpu as pltpu
from jax.experimental.pallas import tpu_sc as plsc

BATCH = 4096
N_FIELDS = 26
N_DENSE = 13
VOCAB = 100000
EMBED = 64
LANES = 16

NC = 2
NS = 16

W = 256
NCHUNK = VOCAB // W + 1
NFULL = NCHUNK - 1
TAIL0 = (VOCAB // 128) * 128
TAIL_MAIN0 = NFULL * W
NROWS = N_FIELDS * BATCH
FLUSH = 128
DUMP0 = NROWS
OUT3_ROWS = NROWS + 32 * FLUSH
OUT_COLS = N_FIELDS * EMBED + N_DENSE
HBUF = 400
NW = NC * NS
TOTALC = N_FIELDS * NCHUNK


def _body(sidx_hbm, tablesT_hbm, tail_hbm, out_hbm,
          vidx_v, sorted_v, hist_v, cur_v, off_s,
          buf0, buf1, buf2, buf3, stag_v, oidx_v,
          sem0, sem1, sem2, sem3, wsem):
    wid = lax.axis_index("s") * NC + lax.axis_index("c")
    lane = lax.iota(jnp.int32, LANES)
    zeros16 = jnp.zeros((LANES,), jnp.int32)
    ones16 = jnp.ones((LANES,), jnp.int32)
    bufs = ((buf0, sem0), (buf1, sem1), (buf2, sem2), (buf3, sem3))

    dump = DUMP0 + wid * FLUSH
    dump_lane = dump + lane

    def _reset_oidx(q16):
        for t in range(FLUSH // LANES):
            plsc.store_scatter(oidx_v, [q16, t * LANES + lane],
                               dump + t * LANES + lane)
    _reset_oidx(zeros16)
    _reset_oidx(ones16)

    def _drain():
        pltpu.make_async_copy(stag_v.at[0], out_hbm.at[oidx_v.at[0]],
                              wsem).wait()

    pltpu.async_copy(stag_v.at[1], out_hbm.at[oidx_v.at[1]], wsem)

    def _flush(p):
        _drain()
        _reset_oidx(jnp.full((LANES,), 1 - p, jnp.int32))
        pltpu.async_copy(stag_v.at[p], out_hbm.at[oidx_v.at[p]], wsem)

    def _segment(f, clo, chi, jp):
        pltpu.sync_copy(
            sidx_hbm.at[pl.ds(pl.multiple_of(f * BATCH, BATCH), BATCH)],
            vidx_v)

        def _issue(c, buf, sem):
            @pl.when(c < NFULL)
            def _():
                col0 = pl.multiple_of(c * W, W)
                pltpu.async_copy(tablesT_hbm.at[f, :, pl.ds(col0, W)],
                                 buf, sem)

            @pl.when(c == NFULL)
            def _():
                pltpu.async_copy(tablesT_hbm.at[f, :, pl.ds(TAIL_MAIN0, 128)],
                                 buf.at[:, pl.ds(0, 128)], sem)
                pltpu.async_copy(tail_hbm.at[f],
                                 buf.at[:, pl.ds(128, 128)], sem)

        def _wait(buf, sem):
            pltpu.make_async_copy(tablesT_hbm.at[f, :, pl.ds(0, W)],
                                  buf, sem).wait()

        for k, (buf, sem) in enumerate(bufs):
            @pl.when(clo + k < chi)
            def _(k=k, buf=buf, sem=sem):
                _issue(clo + k, buf, sem)

        for t in range(HBUF // LANES):
            hist_v[pl.ds(t * LANES, LANES)] = zeros16

        def _hist(t, carry):
            v16 = plsc.load_gather(vidx_v, [t * LANES + lane])
            plsc.addupdate_scatter(hist_v, [v16 >> 8], ones16)
            return carry
        lax.fori_loop(0, BATCH // LANES, _hist, 0)

        carry = jnp.int32(0)
        for t in range(HBUF // LANES):
            h16 = hist_v[pl.ds(t * LANES, LANES)]
            incl = plsc.cumsum(h16)
            excl = incl - h16 + carry
            cur_v[pl.ds(t * LANES, LANES)] = excl
            for i in range(LANES):
                off_s[t * LANES + i] = excl[i]
            carry = carry + incl[LANES - 1]

        def _place(t, carry):
            kv = t * LANES + lane
            v16 = plsc.load_gather(vidx_v, [kv])
            c16 = v16 >> 8
            rec16 = (kv << 8) | (v16 & (W - 1))
            dup16, _last = plsc.scan_count(c16)
            base16 = plsc.load_gather(cur_v, [c16])
            plsc.store_scatter(sorted_v, [base16 + dup16 - 1], rec16)
            plsc.addupdate_scatter(cur_v, [c16], ones16)
            return carry
        lax.fori_loop(0, BATCH // LANES, _place, 0)

        def _process(n0, n1, buf, jp):
            ngroups = (n1 - n0 + LANES - 1) >> 4

            def _group(g, jp):
                j, p = jp
                full = j == FLUSH

                @pl.when(full)
                def _():
                    _flush(p)
                j = lax.select(full, 0, j)
                p = lax.select(full, 1 - p, p)
                p16 = jnp.full((LANES,), p, jnp.int32)
                kv = n0 + g * LANES + lane
                kidx = jnp.minimum(kv, n1 - 1)
                rec16 = plsc.load_gather(sorted_v, [kidx])
                rowid16 = jnp.where(kv < n1, f * BATCH + (rec16 >> 8),
                                    dump_lane)
                plsc.store_scatter(oidx_v, [p16, j + lane], rowid16)
                for i in range(LANES):
                    col = rec16[i] & (W - 1)
                    cols = jnp.full((LANES,), col, jnp.int32)
                    j16 = jnp.full((LANES,), j + i, jnp.int32)
                    for t in range(EMBED // LANES):
                        vals = plsc.load_gather(buf, [lane + t * LANES, cols])
                        plsc.store_scatter(stag_v,
                                           [p16, j16, lane + t * LANES], vals)
                return j + LANES, p
            return lax.fori_loop(0, ngroups, _group, jp)

        ntrips = (chi - clo + 3) // 4

        def _trip(i, jp):
            c0 = clo + 4 * i
            for k, (buf, sem) in enumerate(bufs):
                c = c0 + k
                live = c < chi

                @pl.when(live)
                def _(buf=buf, sem=sem):
                    _wait(buf, sem)
                cc = jnp.minimum(c, NCHUNK - 1)
                n0 = lax.select(live, off_s[cc], 0)
                n1 = lax.select(live, off_s[cc + 1], 0)
                jp = _process(n0, n1, buf, jp)

                @pl.when(c + 4 < chi)
                def _(c=c, buf=buf, sem=sem):
                    _issue(c + 4, buf, sem)
            return jp

        return lax.fori_loop(0, ntrips, _trip, jp)

    g0 = (wid * TOTALC) // NW
    g1 = ((wid + 1) * TOTALC) // NW
    fA = g0 // NCHUNK
    cA0 = g0 % NCHUNK
    fZ = (g1 - 1) // NCHUNK
    cZ1 = (g1 - 1) % NCHUNK + 1
    two = fZ > fA
    eA = lax.select(two, jnp.int32(NCHUNK), cZ1)
    cB0 = jnp.int32(0)
    eB = lax.select(two, cZ1, jnp.int32(0))

    jp = (jnp.int32(0), jnp.int32(0))
    jp = _segment(fA, cA0, eA, jp)
    jp = _segment(fZ, cB0, eB, jp)
    j, p = jp
    _flush(p)
    _drain()


@jax.jit
def _sc_embed(sidxT, tablesT, tailT):
    mesh = plsc.VectorSubcoreMesh(core_axis_name="c", subcore_axis_name="s")
    fn = pl.kernel(
        _body,
        mesh=mesh,
        compiler_params=pltpu.CompilerParams(needs_layout_passes=False),
        out_type=jax.ShapeDtypeStruct((OUT3_ROWS, 2 * EMBED), jnp.float32),
        scratch_types=[
            pltpu.VMEM((BATCH,), jnp.int32),
            pltpu.VMEM((BATCH,), jnp.int32),
            pltpu.VMEM((HBUF,), jnp.int32),
            pltpu.VMEM((HBUF,), jnp.int32),
            pltpu.SMEM((HBUF,), jnp.int32),
            pltpu.VMEM((EMBED, W), jnp.float32),
            pltpu.VMEM((EMBED, W), jnp.float32),
            pltpu.VMEM((EMBED, W), jnp.float32),
            pltpu.VMEM((EMBED, W), jnp.float32),
            pltpu.VMEM((2, FLUSH, 2 * EMBED), jnp.float32),
            pltpu.VMEM((2, FLUSH), jnp.int32),
            pltpu.SemaphoreType.DMA,
            pltpu.SemaphoreType.DMA,
            pltpu.SemaphoreType.DMA,
            pltpu.SemaphoreType.DMA,
            pltpu.SemaphoreType.DMA,
        ],
    )
    return fn(sidxT, tablesT, tailT)


BB = 256


def _asm_body(emb_ref, dense_ref, out_ref):
    for fp in range(N_FIELDS // 2):
        pair = jnp.concatenate(
            [emb_ref[2 * fp, :, :EMBED], emb_ref[2 * fp + 1, :, :EMBED]],
            axis=1)
        out_ref[:, 2 * EMBED * fp:2 * EMBED * (fp + 1)] = pair
    out_ref[:, N_FIELDS * EMBED:] = dense_ref[...]


@jax.jit
def _assemble(emb3, dense):
    return pl.pallas_call(
        _asm_body,
        grid=(BATCH // BB,),
        in_specs=[
            pl.BlockSpec((N_FIELDS + 1, BB, 2 * EMBED), lambda i: (0, i, 0)),
            pl.BlockSpec((BB, N_DENSE), lambda i: (i, 0)),
        ],
        out_specs=pl.BlockSpec((BB, OUT_COLS), lambda i: (i, 0)),
        out_shape=jax.ShapeDtypeStruct((BATCH, OUT_COLS), jnp.float32),
    )(emb3, dense)


def kernel(sparse_indices, dense_values, tables):
    tablesT = tables.transpose(0, 2, 1)
    tailT = tables[:, TAIL0:, :].transpose(0, 2, 1)
    tailT = jnp.pad(tailT, ((0, 0), (0, 0), (0, 128 - (VOCAB - TAIL0))))
    sidxT = sparse_indices.T.reshape(-1)
    out3 = _sc_embed(sidxT, tablesT, tailT)
    emb3 = out3.reshape(N_FIELDS + 1, BATCH, 2 * EMBED)
    return _assemble(emb3, dense_values)
